# baseline (speedup 1.0000x reference)
import functools
import jax
import jax.numpy as jnp
from jax import lax
from jax.experimental import pallas as pl
from jax.experimental.pallas import tpu as pltpu
from jax.experimental.pallas import tpu_sc as plsc

INPUT_DIM = 1000000
OUTPUT_DIM = 64
PAD_DIM = 128
BATCH = 4096
SEQ_LEN = 200
NC, NS = 2, 16
NW = NC * NS
CHUNK = BATCH // NW
NBUF = 5

_mesh = plsc.VectorSubcoreMesh(core_axis_name="c", subcore_axis_name="s")

@functools.partial(
    pl.kernel,
    out_type=jax.ShapeDtypeStruct((BATCH, SEQ_LEN, OUTPUT_DIM), jnp.float32),
    mesh=_mesh,
    scratch_types=[
        pltpu.VMEM((SEQ_LEN, CHUNK), jnp.int32),
        pltpu.VMEM((NBUF, CHUNK, PAD_DIM), jnp.float32),
        pltpu.SemaphoreType.DMA((NBUF,)),
        pltpu.SemaphoreType.DMA((NBUF,)),
    ],
    compiler_params=pltpu.CompilerParams(needs_layout_passes=False, use_tc_tiling_on_sc=False),
)
def _emb_lookup(xt_hbm, w_hbm, out_hbm, idx_v, rows_v, in_sems, out_sems):
    wid = lax.axis_index("s") * NC + lax.axis_index("c")
    b0 = wid * CHUNK
    pltpu.sync_copy(xt_hbm.at[:, pl.ds(b0, CHUNK)], idx_v)

    def start_gather(s, j):
        pltpu.async_copy(w_hbm.at[idx_v.at[s]], rows_v.at[j], in_sems.at[j])

    def wait_gather(s, j):
        pltpu.make_async_copy(w_hbm.at[idx_v.at[s]], rows_v.at[j], in_sems.at[j]).wait()

    def start_write(s, j):
        pltpu.async_copy(
            rows_v.at[j, :, pl.ds(0, OUTPUT_DIM)],
            out_hbm.at[pl.ds(b0, CHUNK), s],
            out_sems.at[j],
        )

    def wait_write(s, j):
        pltpu.make_async_copy(
            rows_v.at[j, :, pl.ds(0, OUTPUT_DIM)],
            out_hbm.at[pl.ds(b0, CHUNK), s],
            out_sems.at[j],
        ).wait()

    for j in range(NBUF):
        start_gather(j, j)

    @pl.loop(0, SEQ_LEN - NBUF, step=NBUF)
    def _ring(s0):
        for j in range(NBUF):
            s = s0 + j
            wait_gather(s, j)
            start_write(s, j)
            wait_write(s, j)
            start_gather(s + NBUF, j)

    for j in range(NBUF):
        s = SEQ_LEN - NBUF + j
        wait_gather(s, j)
        start_write(s, j)
    for j in range(NBUF):
        s = SEQ_LEN - NBUF + j
        wait_write(s, j)


def kernel(x, W):
    xt = x.astype(jnp.int32).T
    w_pad = jnp.pad(W, ((0, 0), (0, PAD_DIM - OUTPUT_DIM)))
    return _emb_lookup(xt, w_pad)


# speedup vs baseline: 1.2331x; 1.2331x over previous
import functools
import jax
import jax.numpy as jnp
from jax import lax
from jax.experimental import pallas as pl
from jax.experimental.pallas import tpu as pltpu
from jax.experimental.pallas import tpu_sc as plsc

INPUT_DIM = 1000000
OUTPUT_DIM = 64
PAD_DIM = 128
BATCH = 4096
SEQ_LEN = 200
NC, NS = 2, 16
NW = NC * NS
CHUNK = BATCH // NW
NBUF = 5

_mesh = plsc.VectorSubcoreMesh(core_axis_name="c", subcore_axis_name="s")

@functools.partial(
    pl.kernel,
    out_type=jax.ShapeDtypeStruct((BATCH, SEQ_LEN, PAD_DIM), jnp.float32),
    mesh=_mesh,
    scratch_types=[
        pltpu.VMEM((SEQ_LEN, CHUNK), jnp.int32),
        pltpu.VMEM((NBUF, CHUNK, PAD_DIM), jnp.float32),
        pltpu.SemaphoreType.DMA((NBUF,)),
        pltpu.SemaphoreType.DMA((NBUF,)),
    ],
    compiler_params=pltpu.CompilerParams(needs_layout_passes=False),
)
def _emb_lookup(xt_hbm, w_hbm, out_hbm, idx_v, rows_v, in_sems, out_sems):
    wid = lax.axis_index("s") * NC + lax.axis_index("c")
    b0 = wid * CHUNK
    pltpu.sync_copy(xt_hbm.at[:, pl.ds(b0, CHUNK)], idx_v)

    def start_gather(s, j):
        pltpu.async_copy(w_hbm.at[idx_v.at[s]], rows_v.at[j], in_sems.at[j])

    def wait_gather(s, j):
        pltpu.make_async_copy(w_hbm.at[idx_v.at[s]], rows_v.at[j], in_sems.at[j]).wait()

    def start_write(s, j):
        pltpu.async_copy(rows_v.at[j], out_hbm.at[pl.ds(b0, CHUNK), s], out_sems.at[j])

    def wait_write(s, j):
        pltpu.make_async_copy(rows_v.at[j], out_hbm.at[pl.ds(b0, CHUNK), s], out_sems.at[j]).wait()

    for j in range(NBUF):
        start_gather(j, j)

    @pl.loop(0, SEQ_LEN - NBUF, step=NBUF)
    def _ring(s0):
        for j in range(NBUF):
            s = s0 + j
            wait_gather(s, j)
            start_write(s, j)
            wait_write(s, j)
            start_gather(s + NBUF, j)

    for j in range(NBUF):
        s = SEQ_LEN - NBUF + j
        wait_gather(s, j)
        start_write(s, j)
    for j in range(NBUF):
        s = SEQ_LEN - NBUF + j
        wait_write(s, j)


def kernel(x, W):
    xt = x.astype(jnp.int32).T
    w_pad = jnp.pad(W, ((0, 0), (0, PAD_DIM - OUTPUT_DIM)))
    out = _emb_lookup(xt, w_pad)
    return out[..., :OUTPUT_DIM]
